# Initial kernel scaffold; baseline (speedup 1.0000x reference)
import jax
import jax.numpy as jnp
from jax.experimental import pallas as pl


def kernel(x, w_qkv2, dw3, pw3, dw5, pw5, mlp_w, mlp_b, proj_w, ln_g, ln_b):
    raise NotImplementedError("write your pallas kernel here")


# speedup vs baseline: 6.7889x; 6.7889x over previous
import functools

import jax
import jax.numpy as jnp
from jax.experimental import pallas as pl
from jax.experimental.pallas import tpu as pltpu

DIMC = 384
D3 = 1152
NWIN = 7
NHW = 49
SHW = 64
TOPK = 4
HEADS = 12
HD = 32
HH = 56
P = HH * HH
PBLK = 448
SCALE = DIMC ** (-0.5)


def _qkv_body(x_ref, w_ref, o_ref):
    o_ref[...] = jnp.dot(x_ref[...], w_ref[...],
                         preferred_element_type=jnp.float32)


def _qkv_call(x_pm, wq):
    rows = x_pm.shape[0]
    return pl.pallas_call(
        _qkv_body,
        grid=(rows // PBLK,),
        in_specs=[
            pl.BlockSpec((PBLK, DIMC), lambda i: (i, 0)),
            pl.BlockSpec((DIMC, D3), lambda i: (0, 0)),
        ],
        out_specs=pl.BlockSpec((PBLK, D3), lambda i: (i, 0)),
        out_shape=jax.ShapeDtypeStruct((rows, D3), jnp.float32),
    )(x_pm, wq)


def _dwpw_body(x_ref, w3_ref, w5_ref, wd3_ref, wd5_ref, a3_ref, a5_ref):
    x = x_ref[0]
    s = [x[:, dx:dx + HH, :] for dx in range(5)]
    def _rowsum(w_ref, kk, ks, off):
        rows = []
        for ky in range(kk):
            r = None
            for kx in range(kk):
                w = w_ref[kk * ky + kx, :]
                term = w[None, None, :] * s[kx + off][ky + off:ky + off + HH]
                r = term if r is None else r + term
            rows.append(r)
        acc = rows[0]
        for r in rows[1:]:
            acc = acc + r
        return acc

    acc3 = _rowsum(w3_ref, 3, s, 1)
    acc5 = _rowsum(w5_ref, 5, s, 0)
    a3_ref[0] = jnp.dot(acc3.astype(jnp.bfloat16).reshape(P, 128),
                        wd3_ref[0].astype(jnp.bfloat16),
                        preferred_element_type=jnp.float32)
    a5_ref[0] = jnp.dot(acc5.astype(jnp.bfloat16).reshape(P, 128),
                        wd5_ref[0].astype(jnp.bfloat16),
                        preferred_element_type=jnp.float32)


def _dwpw_call(qpad, w3r, w5r, wd3, wd5):
    b = qpad.shape[0]
    grid = (b, D3 // 128)
    return pl.pallas_call(
        _dwpw_body,
        grid=grid,
        in_specs=[
            pl.BlockSpec((1, 64, 64, 128), lambda ib, s: (ib, 0, 0, s)),
            pl.BlockSpec((9, 128), lambda ib, s: (0, s)),
            pl.BlockSpec((25, 128), lambda ib, s: (0, s)),
            pl.BlockSpec((1, 128, 128), lambda ib, s: (s, 0, 0)),
            pl.BlockSpec((1, 128, 128), lambda ib, s: (s, 0, 0)),
        ],
        out_specs=[
            pl.BlockSpec((1, P, 128), lambda ib, s: (ib, 0, s)),
            pl.BlockSpec((1, P, 128), lambda ib, s: (ib, 0, s)),
        ],
        out_shape=[
            jax.ShapeDtypeStruct((b, P, D3), jnp.float32),
            jax.ShapeDtypeStruct((b, P, D3), jnp.float32),
        ],
    )(qpad, w3r, w5r, wd3, wd5)


def _route_body(q_ref, k_ref, idx_ref):
    qw = jnp.max(q_ref[0], axis=1) * SCALE
    kw = jnp.max(k_ref[0], axis=1)
    logits = jax.lax.dot_general(qw, kw, (((1,), (1,)), ((), ())),
                                 preferred_element_type=jnp.float32)
    iota = jax.lax.broadcasted_iota(jnp.int32, (NHW, NHW), 1)
    cols = []
    l = logits
    for _ in range(TOPK):
        m = jnp.max(l, axis=1, keepdims=True)
        idx_t = jnp.min(jnp.where(l >= m, iota, NHW), axis=1)
        cols.append(idx_t[:, None])
        l = jnp.where(iota == idx_t[:, None], -jnp.inf, l)
    idx_ref[0] = jnp.concatenate(cols, axis=1)


def _route_call(q_win, kv_win):
    b = q_win.shape[0]
    return pl.pallas_call(
        _route_body,
        grid=(b,),
        in_specs=[
            pl.BlockSpec((1, NHW, SHW, DIMC), lambda ib: (ib, 0, 0, 0)),
            pl.BlockSpec((1, NHW, SHW, DIMC), lambda ib: (ib, 0, 0, 0)),
        ],
        out_specs=pl.BlockSpec((1, NHW, TOPK), lambda ib: (ib, 0, 0)),
        out_shape=jax.ShapeDtypeStruct((b, NHW, TOPK), jnp.int32),
    )(q_win, kv_win)


def _attn_body(ridx_ref, q_ref, kv0_ref, kv1_ref, kv2_ref, kv3_ref, o_ref):
    del ridx_ref
    q = q_ref[0, 0] * SCALE
    kvs = [kv0_ref[0, 0], kv1_ref[0, 0], kv2_ref[0, 0], kv3_ref[0, 0]]
    ks = jnp.concatenate([kv[:, :DIMC] for kv in kvs], axis=0)
    vs = jnp.concatenate([kv[:, DIMC:] for kv in kvs], axis=0)
    outs = []
    for h in range(HEADS):
        sl = slice(h * HD, (h + 1) * HD)
        lg = jax.lax.dot_general(q[:, sl], ks[:, sl],
                                 (((1,), (1,)), ((), ())),
                                 preferred_element_type=jnp.float32)
        m = jnp.max(lg, axis=1, keepdims=True)
        e = jnp.exp(lg - m)
        den = jnp.sum(e, axis=1, keepdims=True)
        outs.append(jnp.dot(e, vs[:, sl],
                            preferred_element_type=jnp.float32) / den)
    o_ref[0, 0] = jnp.concatenate(outs, axis=1)


def _attn_call(r_idx, q_win, kv_win):
    b = q_win.shape[0]
    kv_spec = [
        pl.BlockSpec((1, 1, SHW, 2 * DIMC),
                     functools.partial(
                         lambda ib, iw, ridx, t: (ib, ridx[ib, iw, t], 0, 0),
                         t=t))
        for t in range(TOPK)
    ]
    grid_spec = pltpu.PrefetchScalarGridSpec(
        num_scalar_prefetch=1,
        grid=(b, NHW),
        in_specs=[
            pl.BlockSpec((1, 1, SHW, DIMC), lambda ib, iw, ridx: (ib, iw, 0, 0)),
            *kv_spec,
        ],
        out_specs=pl.BlockSpec((1, 1, SHW, DIMC),
                               lambda ib, iw, ridx: (ib, iw, 0, 0)),
    )
    return pl.pallas_call(
        _attn_body,
        grid_spec=grid_spec,
        out_shape=jax.ShapeDtypeStruct((b, NHW, SHW, DIMC), jnp.float32),
    )(r_idx, q_win, kv_win, kv_win, kv_win, kv_win)


def _out_body(o0_ref, o1_ref, o2_ref, wt_ref, c_ref, g_ref, b_ref, y_ref):
    acc = (jnp.dot(o0_ref[...], wt_ref[0], preferred_element_type=jnp.float32)
           + jnp.dot(o1_ref[...], wt_ref[1], preferred_element_type=jnp.float32)
           + jnp.dot(o2_ref[...], wt_ref[2], preferred_element_type=jnp.float32)
           + c_ref[...])
    mu = jnp.mean(acc, axis=1, keepdims=True)
    var = jnp.mean((acc - mu) ** 2, axis=1, keepdims=True)
    y_ref[...] = (acc - mu) * jax.lax.rsqrt(var + 1e-5) * g_ref[...] + b_ref[...]


def _out_call(o0, o1, o2, wt, cvec, g, bb):
    rows = o0.shape[0]
    row_spec = pl.BlockSpec((PBLK, DIMC), lambda i: (i, 0))
    return pl.pallas_call(
        _out_body,
        grid=(rows // PBLK,),
        in_specs=[
            row_spec, row_spec, row_spec,
            pl.BlockSpec((3, DIMC, DIMC), lambda i: (0, 0, 0)),
            pl.BlockSpec((1, DIMC), lambda i: (0, 0)),
            pl.BlockSpec((1, DIMC), lambda i: (0, 0)),
            pl.BlockSpec((1, DIMC), lambda i: (0, 0)),
        ],
        out_specs=row_spec,
        out_shape=jax.ShapeDtypeStruct((rows, DIMC), jnp.float32),
    )(o0, o1, o2, wt, cvec, g, bb)


def _blockdiag(pw):
    blk = jnp.swapaxes(pw.reshape(36, 32, 32), 1, 2).reshape(9, 4, 32, 32)
    wd = jnp.zeros((9, 128, 128), jnp.float32)
    for j in range(4):
        wd = wd.at[:, j * 32:(j + 1) * 32, j * 32:(j + 1) * 32].set(blk[:, j])
    return wd


def _to_win(t, b):
    c = t.shape[-1]
    return (t.reshape(b, NWIN, 8, NWIN, 8, c)
            .transpose(0, 1, 3, 2, 4, 5)
            .reshape(b, NHW, SHW, c))


def kernel(x, w_qkv2, dw3, pw3, dw5, pw5, mlp_w, mlp_b, proj_w, ln_g, ln_b):
    b = x.shape[0]
    x_pm = x.transpose(0, 2, 3, 1).reshape(b * P, DIMC)
    wq = w_qkv2.reshape(D3, DIMC).T
    qkv = _qkv_call(x_pm, wq)

    qpad = jnp.pad(qkv.reshape(b, HH, HH, D3),
                   ((0, 0), (2, 6), (2, 6), (0, 0)))
    w3r = dw3.reshape(D3, 9).T
    w5r = dw5.reshape(D3, 25).T
    a3, a5 = _dwpw_call(qpad, w3r, w5r, _blockdiag(pw3), _blockdiag(pw5))

    branch_outs = []
    for el in (qkv.reshape(b, P, D3), a3, a5):
        ew = _to_win(el, b)
        q_win = ew[..., :DIMC]
        kv_win = ew[..., DIMC:]
        r_idx = _route_call(q_win, kv_win)
        branch_outs.append(_attn_call(r_idx, q_win, kv_win))

    pmat = proj_w.reshape(DIMC, D3)
    wt = jnp.stack([(pmat[:, i * DIMC:(i + 1) * DIMC] @ mlp_w).T
                    for i in range(3)])
    cvec = (pmat @ jnp.tile(mlp_b, 3)).reshape(1, DIMC)
    y = _out_call(branch_outs[0].reshape(-1, DIMC),
                  branch_outs[1].reshape(-1, DIMC),
                  branch_outs[2].reshape(-1, DIMC),
                  wt, cvec, ln_g.reshape(1, DIMC), ln_b.reshape(1, DIMC))
    return (y.reshape(b, NWIN, NWIN, 8, 8, DIMC)
            .transpose(0, 5, 1, 3, 2, 4)
            .reshape(b, DIMC, HH, HH))


# speedup vs baseline: 7.3298x; 1.0797x over previous
import functools

import jax
import jax.numpy as jnp
from jax.experimental import pallas as pl
from jax.experimental.pallas import tpu as pltpu

DIMC = 384
D3 = 1152
NWIN = 7
NHW = 49
SHW = 64
TOPK = 4
HEADS = 12
HD = 32
HH = 56
P = HH * HH
PBLK = 448
SCALE = DIMC ** (-0.5)


def _win2sp(t):
    c = t.shape[-1]
    return (t.reshape(NWIN, NWIN, 8, 8, c)
            .transpose(0, 2, 1, 3, 4)
            .reshape(HH, HH, c))


def _sp2win(t):
    c = t.shape[-1]
    return (t.reshape(NWIN, 8, NWIN, 8, c)
            .transpose(0, 2, 1, 3, 4)
            .reshape(P, c))


def _qkv_body(x_ref, w_ref, o_ref):
    o_ref[...] = jnp.dot(x_ref[...], w_ref[...],
                         preferred_element_type=jnp.float32)


def _qkv_call(x_pm, wq):
    rows = x_pm.shape[0]
    return pl.pallas_call(
        _qkv_body,
        grid=(rows // PBLK,),
        in_specs=[
            pl.BlockSpec((PBLK, DIMC), lambda i: (i, 0)),
            pl.BlockSpec((DIMC, D3), lambda i: (0, 0)),
        ],
        out_specs=pl.BlockSpec((PBLK, D3), lambda i: (i, 0)),
        out_shape=jax.ShapeDtypeStruct((rows, D3), jnp.float32),
    )(x_pm, wq)


def _dwpw_body(x_ref, w3_ref, w5_ref, wd3_ref, wd5_ref, a3_ref, a5_ref):
    xs = _win2sp(x_ref[0])
    x = jnp.pad(xs, ((2, 6), (2, 6), (0, 0)))
    s = [x[:, dx:dx + HH, :] for dx in range(5)]
    acc3 = jnp.zeros((HH, HH, 128), jnp.float32)
    for ky in range(3):
        for kx in range(3):
            w = w3_ref[3 * ky + kx, :]
            acc3 = acc3 + w[None, None, :] * s[kx + 1][ky + 1:ky + 1 + HH]
    acc5 = jnp.zeros((HH, HH, 128), jnp.float32)
    for ky in range(5):
        for kx in range(5):
            w = w5_ref[5 * ky + kx, :]
            acc5 = acc5 + w[None, None, :] * s[kx][ky:ky + HH]
    a3_ref[0] = jnp.dot(_sp2win(acc3), wd3_ref[0],
                        preferred_element_type=jnp.float32)
    a5_ref[0] = jnp.dot(_sp2win(acc5), wd5_ref[0],
                        preferred_element_type=jnp.float32)


def _dwpw_call(qkv_w, w3r, w5r, wd3, wd5):
    b = qkv_w.shape[0]
    grid = (b, D3 // 128)
    return pl.pallas_call(
        _dwpw_body,
        grid=grid,
        in_specs=[
            pl.BlockSpec((1, P, 128), lambda ib, s: (ib, 0, s)),
            pl.BlockSpec((9, 128), lambda ib, s: (0, s)),
            pl.BlockSpec((25, 128), lambda ib, s: (0, s)),
            pl.BlockSpec((1, 128, 128), lambda ib, s: (s, 0, 0)),
            pl.BlockSpec((1, 128, 128), lambda ib, s: (s, 0, 0)),
        ],
        out_specs=[
            pl.BlockSpec((1, P, 128), lambda ib, s: (ib, 0, s)),
            pl.BlockSpec((1, P, 128), lambda ib, s: (ib, 0, s)),
        ],
        out_shape=[
            jax.ShapeDtypeStruct((b, P, D3), jnp.float32),
            jax.ShapeDtypeStruct((b, P, D3), jnp.float32),
        ],
    )(qkv_w, w3r, w5r, wd3, wd5)


def _route_body(q_ref, k_ref, idx_ref):
    qw = jnp.max(q_ref[0], axis=1) * SCALE
    kw = jnp.max(k_ref[0], axis=1)
    logits = jax.lax.dot_general(qw, kw, (((1,), (1,)), ((), ())),
                                 preferred_element_type=jnp.float32)
    iota = jax.lax.broadcasted_iota(jnp.int32, (NHW, NHW), 1)
    cols = []
    l = logits
    for _ in range(TOPK):
        m = jnp.max(l, axis=1, keepdims=True)
        idx_t = jnp.min(jnp.where(l >= m, iota, NHW), axis=1)
        cols.append(idx_t[:, None])
        l = jnp.where(iota == idx_t[:, None], -jnp.inf, l)
    idx_ref[0] = jnp.concatenate(cols, axis=1)


def _route_call(ew):
    b = ew.shape[0]
    return pl.pallas_call(
        _route_body,
        grid=(b,),
        in_specs=[
            pl.BlockSpec((1, NHW, SHW, DIMC), lambda ib: (ib, 0, 0, 0)),
            pl.BlockSpec((1, NHW, SHW, DIMC), lambda ib: (ib, 0, 0, 1)),
        ],
        out_specs=pl.BlockSpec((1, NHW, TOPK), lambda ib: (ib, 0, 0)),
        out_shape=jax.ShapeDtypeStruct((b, NHW, TOPK), jnp.int32),
    )(ew, ew)


def _attn_body(ridx_ref, q_ref, k0_ref, k1_ref, k2_ref, k3_ref,
               v0_ref, v1_ref, v2_ref, v3_ref, o_ref):
    del ridx_ref
    q = q_ref[0, 0] * SCALE
    ks = jnp.concatenate([k0_ref[0, 0], k1_ref[0, 0],
                          k2_ref[0, 0], k3_ref[0, 0]], axis=0)
    vs = jnp.concatenate([v0_ref[0, 0], v1_ref[0, 0],
                          v2_ref[0, 0], v3_ref[0, 0]], axis=0)
    outs = []
    for h in range(HEADS):
        sl = slice(h * HD, (h + 1) * HD)
        lg = jax.lax.dot_general(q[:, sl], ks[:, sl],
                                 (((1,), (1,)), ((), ())),
                                 preferred_element_type=jnp.float32)
        m = jnp.max(lg, axis=1, keepdims=True)
        e = jnp.exp(lg - m)
        den = jnp.sum(e, axis=1, keepdims=True)
        outs.append(jnp.dot(e, vs[:, sl],
                            preferred_element_type=jnp.float32) / den)
    o_ref[0, 0] = jnp.concatenate(outs, axis=1)


def _attn_call(r_idx, ew):
    b = ew.shape[0]
    k_spec = [
        pl.BlockSpec((1, 1, SHW, DIMC),
                     functools.partial(
                         lambda ib, iw, ridx, t: (ib, ridx[ib, iw, t], 0, 1),
                         t=t))
        for t in range(TOPK)
    ]
    v_spec = [
        pl.BlockSpec((1, 1, SHW, DIMC),
                     functools.partial(
                         lambda ib, iw, ridx, t: (ib, ridx[ib, iw, t], 0, 2),
                         t=t))
        for t in range(TOPK)
    ]
    grid_spec = pltpu.PrefetchScalarGridSpec(
        num_scalar_prefetch=1,
        grid=(b, NHW),
        in_specs=[
            pl.BlockSpec((1, 1, SHW, DIMC), lambda ib, iw, ridx: (ib, iw, 0, 0)),
            *k_spec,
            *v_spec,
        ],
        out_specs=pl.BlockSpec((1, 1, SHW, DIMC),
                               lambda ib, iw, ridx: (ib, iw, 0, 0)),
    )
    return pl.pallas_call(
        _attn_body,
        grid_spec=grid_spec,
        out_shape=jax.ShapeDtypeStruct((b, NHW, SHW, DIMC), jnp.float32),
    )(r_idx, *([ew] * 9))


def _out_body(o0_ref, o1_ref, o2_ref, wt_ref, c_ref, g_ref, b_ref, y_ref):
    acc = (jnp.dot(o0_ref[...], wt_ref[0], preferred_element_type=jnp.float32)
           + jnp.dot(o1_ref[...], wt_ref[1], preferred_element_type=jnp.float32)
           + jnp.dot(o2_ref[...], wt_ref[2], preferred_element_type=jnp.float32)
           + c_ref[...])
    mu = jnp.mean(acc, axis=1, keepdims=True)
    var = jnp.mean((acc - mu) ** 2, axis=1, keepdims=True)
    y_ref[...] = (acc - mu) * jax.lax.rsqrt(var + 1e-5) * g_ref[...] + b_ref[...]


def _out_call(o0, o1, o2, wt, cvec, g, bb):
    rows = o0.shape[0]
    row_spec = pl.BlockSpec((PBLK, DIMC), lambda i: (i, 0))
    return pl.pallas_call(
        _out_body,
        grid=(rows // PBLK,),
        in_specs=[
            row_spec, row_spec, row_spec,
            pl.BlockSpec((3, DIMC, DIMC), lambda i: (0, 0, 0)),
            pl.BlockSpec((1, DIMC), lambda i: (0, 0)),
            pl.BlockSpec((1, DIMC), lambda i: (0, 0)),
            pl.BlockSpec((1, DIMC), lambda i: (0, 0)),
        ],
        out_specs=row_spec,
        out_shape=jax.ShapeDtypeStruct((rows, DIMC), jnp.float32),
    )(o0, o1, o2, wt, cvec, g, bb)


def _blockdiag(pw):
    blk = jnp.swapaxes(pw.reshape(36, 32, 32), 1, 2).reshape(9, 4, 32, 32)
    wd = jnp.zeros((9, 128, 128), jnp.float32)
    for j in range(4):
        wd = wd.at[:, j * 32:(j + 1) * 32, j * 32:(j + 1) * 32].set(blk[:, j])
    return wd


def kernel(x, w_qkv2, dw3, pw3, dw5, pw5, mlp_w, mlp_b, proj_w, ln_g, ln_b):
    b = x.shape[0]
    x_win = (x.transpose(0, 2, 3, 1)
             .reshape(b, NWIN, 8, NWIN, 8, DIMC)
             .transpose(0, 1, 3, 2, 4, 5)
             .reshape(b * P, DIMC))
    wq = w_qkv2.reshape(D3, DIMC).T
    qkv = _qkv_call(x_win, wq)

    w3r = dw3.reshape(D3, 9).T
    w5r = dw5.reshape(D3, 25).T
    a3, a5 = _dwpw_call(qkv.reshape(b, P, D3), w3r, w5r,
                        _blockdiag(pw3), _blockdiag(pw5))

    branch_outs = []
    for el in (qkv.reshape(b, P, D3), a3, a5):
        ew = el.reshape(b, NHW, SHW, D3)
        r_idx = _route_call(ew)
        branch_outs.append(_attn_call(r_idx, ew))

    pmat = proj_w.reshape(DIMC, D3)
    wt = jnp.stack([(pmat[:, i * DIMC:(i + 1) * DIMC] @ mlp_w).T
                    for i in range(3)])
    cvec = (pmat @ jnp.tile(mlp_b, 3)).reshape(1, DIMC)
    y = _out_call(branch_outs[0].reshape(-1, DIMC),
                  branch_outs[1].reshape(-1, DIMC),
                  branch_outs[2].reshape(-1, DIMC),
                  wt, cvec, ln_g.reshape(1, DIMC), ln_b.reshape(1, DIMC))
    return (y.reshape(b, NWIN, NWIN, 8, 8, DIMC)
            .transpose(0, 5, 1, 3, 2, 4)
            .reshape(b, DIMC, HH, HH))


# speedup vs baseline: 7.5766x; 1.0337x over previous
import functools

import jax
import jax.numpy as jnp
from jax.experimental import pallas as pl
from jax.experimental.pallas import tpu as pltpu

DIMC = 384
D3 = 1152
NWIN = 7
NHW = 49
SHW = 64
TOPK = 4
HEADS = 12
HD = 32
HH = 56
P = HH * HH
PBLK = 448
SCALE = DIMC ** (-0.5)


def _win2sp(t):
    c = t.shape[-1]
    return (t.reshape(NWIN, NWIN, 8, 8, c)
            .transpose(0, 2, 1, 3, 4)
            .reshape(HH, HH, c))


def _sp2win(t):
    c = t.shape[-1]
    return (t.reshape(NWIN, 8, NWIN, 8, c)
            .transpose(0, 2, 1, 3, 4)
            .reshape(P, c))


def _qkv_body(x_ref, w_ref, o_ref):
    o_ref[...] = jnp.dot(x_ref[...], w_ref[...],
                         preferred_element_type=jnp.float32)


def _qkv_call(x_pm, wq):
    rows = x_pm.shape[0]
    return pl.pallas_call(
        _qkv_body,
        grid=(rows // PBLK,),
        in_specs=[
            pl.BlockSpec((PBLK, DIMC), lambda i: (i, 0)),
            pl.BlockSpec((DIMC, D3), lambda i: (0, 0)),
        ],
        out_specs=pl.BlockSpec((PBLK, D3), lambda i: (i, 0)),
        out_shape=jax.ShapeDtypeStruct((rows, D3), jnp.float32),
    )(x_pm, wq)


def _dwpw_body(x_ref, w3_ref, w5_ref, wd3_ref, wd5_ref, a3_ref, a5_ref):
    xs = _win2sp(x_ref[0])
    x = jnp.pad(xs, ((2, 6), (2, 6), (0, 0)))
    s = [x[:, dx:dx + HH, :] for dx in range(5)]
    acc3 = jnp.zeros((HH, HH, 128), jnp.float32)
    for ky in range(3):
        for kx in range(3):
            w = w3_ref[3 * ky + kx, :]
            acc3 = acc3 + w[None, None, :] * s[kx + 1][ky + 1:ky + 1 + HH]
    acc5 = jnp.zeros((HH, HH, 128), jnp.float32)
    for ky in range(5):
        for kx in range(5):
            w = w5_ref[5 * ky + kx, :]
            acc5 = acc5 + w[None, None, :] * s[kx][ky:ky + HH]
    a3_ref[0] = jnp.dot(_sp2win(acc3), wd3_ref[0],
                        preferred_element_type=jnp.float32)
    a5_ref[0] = jnp.dot(_sp2win(acc5), wd5_ref[0],
                        preferred_element_type=jnp.float32)


def _dwpw_call(qkv_w, w3r, w5r, wd3, wd5):
    b = qkv_w.shape[0]
    grid = (b, D3 // 128)
    return pl.pallas_call(
        _dwpw_body,
        grid=grid,
        in_specs=[
            pl.BlockSpec((1, P, 128), lambda ib, s: (ib, 0, s)),
            pl.BlockSpec((9, 128), lambda ib, s: (0, s)),
            pl.BlockSpec((25, 128), lambda ib, s: (0, s)),
            pl.BlockSpec((1, 128, 128), lambda ib, s: (s, 0, 0)),
            pl.BlockSpec((1, 128, 128), lambda ib, s: (s, 0, 0)),
        ],
        out_specs=[
            pl.BlockSpec((1, P, 128), lambda ib, s: (ib, 0, s)),
            pl.BlockSpec((1, P, 128), lambda ib, s: (ib, 0, s)),
        ],
        out_shape=[
            jax.ShapeDtypeStruct((b, P, D3), jnp.float32),
            jax.ShapeDtypeStruct((b, P, D3), jnp.float32),
        ],
    )(qkv_w, w3r, w5r, wd3, wd5)


def _route_body(q_ref, k_ref, idx_ref):
    qw = jnp.max(q_ref[0], axis=1) * SCALE
    kw = jnp.max(k_ref[0], axis=1)
    logits = jax.lax.dot_general(qw, kw, (((1,), (1,)), ((), ())),
                                 preferred_element_type=jnp.float32)
    iota = jax.lax.broadcasted_iota(jnp.int32, (NHW, NHW), 1)
    cols = []
    l = logits
    for _ in range(TOPK):
        m = jnp.max(l, axis=1, keepdims=True)
        idx_t = jnp.min(jnp.where(l >= m, iota, NHW), axis=1)
        cols.append(idx_t[:, None])
        l = jnp.where(iota == idx_t[:, None], -jnp.inf, l)
    idx_ref[0] = jnp.concatenate(cols, axis=1)


def _route_call(ew):
    b = ew.shape[0]
    return pl.pallas_call(
        _route_body,
        grid=(b,),
        in_specs=[
            pl.BlockSpec((1, NHW, SHW, DIMC), lambda ib: (ib, 0, 0, 2)),
            pl.BlockSpec((1, NHW, SHW, DIMC), lambda ib: (ib, 0, 0, 0)),
        ],
        out_specs=pl.BlockSpec((1, NHW, TOPK), lambda ib: (ib, 0, 0)),
        out_shape=jax.ShapeDtypeStruct((b, NHW, TOPK), jnp.int32),
    )(ew, ew)


def _attn_body(ridx_ref, q_ref, kv_ref, o_ref):
    ib = pl.program_id(0)
    iw = pl.program_id(1)
    q = q_ref[0, 0] * SCALE
    kvs = [kv_ref[0, ridx_ref[ib, iw, t]] for t in range(TOPK)]
    ks = jnp.concatenate([kv[:, :DIMC] for kv in kvs], axis=0)
    vs = jnp.concatenate([kv[:, DIMC:] for kv in kvs], axis=0)
    outs = []
    for h in range(HEADS):
        sl = slice(h * HD, (h + 1) * HD)
        lg = jax.lax.dot_general(q[:, sl], ks[:, sl],
                                 (((1,), (1,)), ((), ())),
                                 preferred_element_type=jnp.float32)
        m = jnp.max(lg, axis=1, keepdims=True)
        e = jnp.exp(lg - m)
        den = jnp.sum(e, axis=1, keepdims=True)
        outs.append(jnp.dot(e, vs[:, sl],
                            preferred_element_type=jnp.float32) / den)
    o_ref[0, 0] = jnp.concatenate(outs, axis=1)


def _attn_call(r_idx, ew):
    b = ew.shape[0]
    grid_spec = pltpu.PrefetchScalarGridSpec(
        num_scalar_prefetch=1,
        grid=(b, NHW),
        in_specs=[
            pl.BlockSpec((1, 1, SHW, DIMC), lambda ib, iw, ridx: (ib, iw, 0, 2)),
            pl.BlockSpec((1, NHW, SHW, 2 * DIMC),
                         lambda ib, iw, ridx: (ib, 0, 0, 0)),
        ],
        out_specs=pl.BlockSpec((1, 1, SHW, DIMC),
                               lambda ib, iw, ridx: (ib, iw, 0, 0)),
    )
    return pl.pallas_call(
        _attn_body,
        grid_spec=grid_spec,
        out_shape=jax.ShapeDtypeStruct((b, NHW, SHW, DIMC), jnp.float32),
    )(r_idx, ew, ew)


def _out_body(o0_ref, o1_ref, o2_ref, wt_ref, c_ref, g_ref, b_ref, y_ref):
    acc = (jnp.dot(o0_ref[...], wt_ref[0], preferred_element_type=jnp.float32)
           + jnp.dot(o1_ref[...], wt_ref[1], preferred_element_type=jnp.float32)
           + jnp.dot(o2_ref[...], wt_ref[2], preferred_element_type=jnp.float32)
           + c_ref[...])
    mu = jnp.mean(acc, axis=1, keepdims=True)
    var = jnp.mean((acc - mu) ** 2, axis=1, keepdims=True)
    y_ref[...] = (acc - mu) * jax.lax.rsqrt(var + 1e-5) * g_ref[...] + b_ref[...]


def _out_call(o0, o1, o2, wt, cvec, g, bb):
    rows = o0.shape[0]
    row_spec = pl.BlockSpec((PBLK, DIMC), lambda i: (i, 0))
    return pl.pallas_call(
        _out_body,
        grid=(rows // PBLK,),
        in_specs=[
            row_spec, row_spec, row_spec,
            pl.BlockSpec((3, DIMC, DIMC), lambda i: (0, 0, 0)),
            pl.BlockSpec((1, DIMC), lambda i: (0, 0)),
            pl.BlockSpec((1, DIMC), lambda i: (0, 0)),
            pl.BlockSpec((1, DIMC), lambda i: (0, 0)),
        ],
        out_specs=row_spec,
        out_shape=jax.ShapeDtypeStruct((rows, DIMC), jnp.float32),
    )(o0, o1, o2, wt, cvec, g, bb)


def _blockdiag(pw):
    blk = jnp.swapaxes(pw.reshape(36, 32, 32), 1, 2).reshape(9, 4, 32, 32)
    wd = jnp.zeros((9, 128, 128), jnp.float32)
    for j in range(4):
        wd = wd.at[:, j * 32:(j + 1) * 32, j * 32:(j + 1) * 32].set(blk[:, j])
    return wd


def kernel(x, w_qkv2, dw3, pw3, dw5, pw5, mlp_w, mlp_b, proj_w, ln_g, ln_b):
    b = x.shape[0]
    x_win = (x.transpose(0, 2, 3, 1)
             .reshape(b, NWIN, 8, NWIN, 8, DIMC)
             .transpose(0, 1, 3, 2, 4, 5)
             .reshape(b * P, DIMC))
    wq = jnp.roll(w_qkv2.reshape(D3, DIMC).T, -DIMC, axis=1)
    qkv = _qkv_call(x_win, wq)

    w3r = jnp.roll(dw3.reshape(D3, 9).T, -DIMC, axis=1)
    w5r = jnp.roll(dw5.reshape(D3, 25).T, -DIMC, axis=1)
    a3, a5 = _dwpw_call(qkv.reshape(b, P, D3), w3r, w5r,
                        jnp.roll(_blockdiag(pw3), -3, axis=0),
                        jnp.roll(_blockdiag(pw5), -3, axis=0))

    branch_outs = []
    for el in (qkv.reshape(b, P, D3), a3, a5):
        ew = el.reshape(b, NHW, SHW, D3)
        r_idx = _route_call(ew)
        branch_outs.append(_attn_call(r_idx, ew))

    pmat = proj_w.reshape(DIMC, D3)
    wt = jnp.stack([(pmat[:, i * DIMC:(i + 1) * DIMC] @ mlp_w).T
                    for i in range(3)])
    cvec = (pmat @ jnp.tile(mlp_b, 3)).reshape(1, DIMC)
    y = _out_call(branch_outs[0].reshape(-1, DIMC),
                  branch_outs[1].reshape(-1, DIMC),
                  branch_outs[2].reshape(-1, DIMC),
                  wt, cvec, ln_g.reshape(1, DIMC), ln_b.reshape(1, DIMC))
    return (y.reshape(b, NWIN, NWIN, 8, 8, DIMC)
            .transpose(0, 5, 1, 3, 2, 4)
            .reshape(b, DIMC, HH, HH))
